# baseline (speedup 1.0000x reference)
import functools

import jax
import jax.numpy as jnp
from jax import lax
from jax.experimental import pallas as pl
from jax.experimental.pallas import tpu as pltpu
from jax.experimental.pallas import tpu_sc as plsc

B, P, H, W, D, C = 4, 8, 32, 32, 32, 32
VOL = H * W * D
N_ROWS = B * P * VOL
NW = 32
CHUNK = 128
N_CHUNKS = VOL // CHUNK
L = 16
CW = C // 2

_POFFS = (0, 1024, 32, 1056)


def _floor_i32(v):
    t = v.astype(jnp.int32)
    return jnp.where(v < t.astype(jnp.float32), t - 1, t)


def _body(table, theta, out, th_v, *rest):
    idx = rest[0:4]
    buf = rest[4:8]
    wbf = rest[8:16]
    outb = rest[16]
    gsem = rest[17]

    wid = lax.axis_index("s") * 2 + lax.axis_index("c")
    pltpu.sync_copy(theta.at[wid], th_v)
    tv = th_v[...]
    t = [tv[i] for i in range(12)]
    vol_base = wid * VOL
    iota = lax.iota(jnp.int32, L)

    def stage(ch, off):
        row0 = ch * CHUNK
        for g in range(CHUNK // L):
            sl = pl.ds(off + g * L, L)
            n = row0 + g * L + iota
            df = (n & 31).astype(jnp.float32)
            wf = ((n >> 5) & 31).astype(jnp.float32)
            hf = (n >> 10).astype(jnp.float32)
            ys = t[0] * hf + t[1] * wf + t[2] * df + t[3]
            xs = t[4] * hf + t[5] * wf + t[6] * df + t[7]
            zs = t[8] * hf + t[9] * wf + t[10] * df + t[11]
            y0 = _floor_i32(ys)
            x0 = _floor_i32(xs)
            z0 = _floor_i32(zs)
            oob = ((x0 < 0) | (x0 >= 31) | (y0 < 0) | (y0 >= 31)
                   | (z0 < 0) | (z0 >= 31))
            y0 = jnp.where(oob, 0, y0)
            x0 = jnp.where(oob, 0, x0)
            z0 = jnp.where(oob, 0, z0)
            base = vol_base + y0 * 1024 + x0 * 32 + z0
            for k in range(4):
                idx[k][sl] = base + _POFFS[k]
            xd = xs - x0.astype(jnp.float32)
            yd = ys - y0.astype(jnp.float32)
            zd = zs - z0.astype(jnp.float32)
            ax, ay, az = 1.0 - xd, 1.0 - yd, 1.0 - zd
            p00, p01 = ax * ay, ax * yd
            p10, p11 = xd * ay, xd * yd
            wbf[0][sl] = p00 * az
            wbf[1][sl] = p00 * zd
            wbf[2][sl] = p01 * az
            wbf[3][sl] = p01 * zd
            wbf[4][sl] = p10 * az
            wbf[5][sl] = p10 * zd
            wbf[6][sl] = p11 * az
            wbf[7][sl] = p11 * zd

    def fire(off):
        for k in range(4):
            pltpu.async_copy(table.at[idx[k].at[pl.ds(off, CHUNK)]],
                             buf[k].at[pl.ds(off, CHUNK)], gsem)

    def drain(off):
        for k in range(4):
            pltpu.make_async_copy(table.at[idx[k].at[pl.ds(off, CHUNK)]],
                                  buf[k].at[pl.ds(off, CHUNK)], gsem).wait()

    stage(0, 0)
    fire(0)

    def chunk_body(ch, carry):
        off = (ch & 1) * CHUNK
        offn = CHUNK - off
        with jax.named_scope("drain_gather"):
            drain(off)
        with jax.named_scope("stage_idx"):
            stage(ch + 1, offn)
        with jax.named_scope("fire_gather"):
            fire(offn)
        with jax.named_scope("blend"):
            def emit(step):
                cw, ws, orow = step
                unp = [plsc.unpack(w, format=plsc.PackFormat.INTERLEAVED,
                                   preferred_element_type=jnp.float32)
                       for w in cw]
                lo = [u[0] for u in unp]
                hi = [u[1] for u in unp]
                for h, c in ((0, lo), (1, hi)):
                    t01 = ws[0] * c[0] + ws[1] * c[1]
                    t23 = ws[2] * c[2] + ws[3] * c[3]
                    t45 = ws[4] * c[4] + ws[5] * c[5]
                    t67 = ws[6] * c[6] + ws[7] * c[7]
                    outb[orow, pl.ds(h * L, L)] = (t01 + t23) + (t45 + t67)

            pend = None
            for g in range(CHUNK // L):
                gsl = pl.ds(off + g * L, L)
                wv = [wbf[k][gsl] for k in range(8)]
                for rl in range(L):
                    row = off + g * L + rl
                    orow = g * L + rl
                    ws = [wv[k][rl] for k in range(8)]
                    cw = [buf[k][row, pl.ds(z * C, C)]
                          for k in range(4) for z in (0, 1)]
                    if pend is not None:
                        emit(pend)
                    pend = (cw, ws, orow)
            emit(pend)
        with jax.named_scope("out_copy"):
            pltpu.sync_copy(outb, out.at[pl.ds(vol_base + ch * CHUNK, CHUNK)])
        return carry

    lax.fori_loop(0, N_CHUNKS, chunk_body, 0)
    drain(0 if N_CHUNKS % 2 == 0 else CHUNK)


_resample = functools.partial(
    pl.kernel,
    mesh=plsc.VectorSubcoreMesh(core_axis_name="c", subcore_axis_name="s"),
    compiler_params=pltpu.CompilerParams(use_tc_tiling_on_sc=False,
                                         needs_layout_passes=False),
    out_type=jax.ShapeDtypeStruct((N_ROWS, C), jnp.float32),
    scratch_types=(
        [pltpu.VMEM((L,), jnp.float32)]
        + [pltpu.VMEM((2 * CHUNK,), jnp.int32) for _ in range(4)]
        + [pltpu.VMEM((2 * CHUNK, 2 * C), jnp.bfloat16) for _ in range(4)]
        + [pltpu.VMEM((2 * CHUNK,), jnp.float32) for _ in range(8)]
        + [pltpu.VMEM((CHUNK, C), jnp.float32),
           pltpu.SemaphoreType.DMA]
    ),
)(_body)

_CHAN_PERM = tuple(c for i in range(CW) for c in (i, i + CW))


def kernel(input_fmap, theta):
    flat = input_fmap.reshape(N_ROWS, C)
    rows = flat[:, jnp.array(_CHAN_PERM)].astype(jnp.bfloat16)
    nxt = jnp.concatenate([rows[1:], jnp.zeros((1, C), jnp.bfloat16)], axis=0)
    packed = jnp.concatenate([rows, nxt], axis=1)
    th = theta.astype(jnp.float32).reshape(NW, 12)
    th = jnp.pad(th, ((0, 0), (0, 4)))
    out = _resample(packed, th)
    return out.reshape(B, P, H, W, D, C)


# speedup vs baseline: 1.2239x; 1.2239x over previous
import functools

import jax
import jax.numpy as jnp
from jax import lax
from jax.experimental import pallas as pl
from jax.experimental.pallas import tpu as pltpu
from jax.experimental.pallas import tpu_sc as plsc

B, P, H, W, D, C = 4, 8, 32, 32, 32, 32
VOL = H * W * D
N_ROWS = B * P * VOL
NW = 32
CHUNK = 128
N_CHUNKS = VOL // CHUNK
L = 16
CW = C // 2

_OFFS = (0, 1, 1024, 1025, 32, 33, 1056, 1057)


def _floor_i32(v):
    t = v.astype(jnp.int32)
    return jnp.where(v < t.astype(jnp.float32), t - 1, t)


def _body(table, theta, out, th_v, *rest):
    idx = rest[0:8]
    buf = rest[8:16]
    wbf = rest[16:24]
    outb = rest[24]
    gsem = rest[25]

    wid = lax.axis_index("s") * 2 + lax.axis_index("c")
    pltpu.sync_copy(theta.at[wid], th_v)
    tv = th_v[...]
    t = [tv[i] for i in range(12)]
    vol_base = wid * VOL
    iota = lax.iota(jnp.int32, L)

    def stage(ch, off):
        row0 = ch * CHUNK
        for g in range(CHUNK // L):
            sl = pl.ds(off + g * L, L)
            n = row0 + g * L + iota
            df = (n & 31).astype(jnp.float32)
            wf = ((n >> 5) & 31).astype(jnp.float32)
            hf = (n >> 10).astype(jnp.float32)
            ys = t[0] * hf + t[1] * wf + t[2] * df + t[3]
            xs = t[4] * hf + t[5] * wf + t[6] * df + t[7]
            zs = t[8] * hf + t[9] * wf + t[10] * df + t[11]
            y0 = _floor_i32(ys)
            x0 = _floor_i32(xs)
            z0 = _floor_i32(zs)
            oob = ((x0 < 0) | (x0 >= 31) | (y0 < 0) | (y0 >= 31)
                   | (z0 < 0) | (z0 >= 31))
            y0 = jnp.where(oob, 0, y0)
            x0 = jnp.where(oob, 0, x0)
            z0 = jnp.where(oob, 0, z0)
            base = vol_base + y0 * 1024 + x0 * 32 + z0
            for k in range(8):
                idx[k][sl] = base + _OFFS[k]
            xd = xs - x0.astype(jnp.float32)
            yd = ys - y0.astype(jnp.float32)
            zd = zs - z0.astype(jnp.float32)
            ax, ay, az = 1.0 - xd, 1.0 - yd, 1.0 - zd
            p00, p01 = ax * ay, ax * yd
            p10, p11 = xd * ay, xd * yd
            wbf[0][sl] = p00 * az
            wbf[1][sl] = p00 * zd
            wbf[2][sl] = p01 * az
            wbf[3][sl] = p01 * zd
            wbf[4][sl] = p10 * az
            wbf[5][sl] = p10 * zd
            wbf[6][sl] = p11 * az
            wbf[7][sl] = p11 * zd

    def fire(off):
        for k in range(8):
            pltpu.async_copy(table.at[idx[k].at[pl.ds(off, CHUNK)]],
                             buf[k].at[pl.ds(off, CHUNK)], gsem)

    def drain(off):
        for k in range(8):
            pltpu.make_async_copy(table.at[idx[k].at[pl.ds(off, CHUNK)]],
                                  buf[k].at[pl.ds(off, CHUNK)], gsem).wait()

    stage(0, 0)
    fire(0)

    def chunk_body(ch, carry):
        off = (ch & 1) * CHUNK
        offn = CHUNK - off
        with jax.named_scope("drain_gather"):
            drain(off)
        with jax.named_scope("stage_idx"):
            stage(ch + 1, offn)
        with jax.named_scope("fire_gather"):
            fire(offn)
        with jax.named_scope("blend"):
            def emit(step):
                cw, ws, orow = step
                unp = [plsc.unpack(w, format=plsc.PackFormat.INTERLEAVED,
                                   preferred_element_type=jnp.float32)
                       for w in cw]
                lo = [u[0] for u in unp]
                hi = [u[1] for u in unp]
                for h, c in ((0, lo), (1, hi)):
                    t01 = ws[0] * c[0] + ws[1] * c[1]
                    t23 = ws[2] * c[2] + ws[3] * c[3]
                    t45 = ws[4] * c[4] + ws[5] * c[5]
                    t67 = ws[6] * c[6] + ws[7] * c[7]
                    outb[orow, pl.ds(h * L, L)] = (t01 + t23) + (t45 + t67)

            pend = None
            for g in range(CHUNK // L):
                gsl = pl.ds(off + g * L, L)
                wv = [wbf[k][gsl] for k in range(8)]
                for rl in range(L):
                    row = off + g * L + rl
                    orow = g * L + rl
                    ws = [wv[k][rl] for k in range(8)]
                    cw = [buf[k][row, :] for k in range(8)]
                    if pend is not None:
                        emit(pend)
                    pend = (cw, ws, orow)
            emit(pend)
        with jax.named_scope("out_copy"):
            pltpu.sync_copy(outb, out.at[pl.ds(vol_base + ch * CHUNK, CHUNK)])
        return carry

    lax.fori_loop(0, N_CHUNKS, chunk_body, 0)
    drain(0 if N_CHUNKS % 2 == 0 else CHUNK)


_resample = functools.partial(
    pl.kernel,
    mesh=plsc.VectorSubcoreMesh(core_axis_name="c", subcore_axis_name="s"),
    compiler_params=pltpu.CompilerParams(use_tc_tiling_on_sc=False,
                                         needs_layout_passes=False),
    out_type=jax.ShapeDtypeStruct((N_ROWS, C), jnp.float32),
    scratch_types=(
        [pltpu.VMEM((L,), jnp.float32)]
        + [pltpu.VMEM((2 * CHUNK,), jnp.int32) for _ in range(8)]
        + [pltpu.VMEM((2 * CHUNK, C), jnp.bfloat16) for _ in range(8)]
        + [pltpu.VMEM((2 * CHUNK,), jnp.float32) for _ in range(8)]
        + [pltpu.VMEM((CHUNK, C), jnp.float32),
           pltpu.SemaphoreType.DMA]
    ),
)(_body)

_CHAN_PERM = tuple(c for i in range(CW) for c in (i, i + CW))


def kernel(input_fmap, theta):
    flat = input_fmap.reshape(N_ROWS, C)
    packed = flat[:, jnp.array(_CHAN_PERM)].astype(jnp.bfloat16)
    th = theta.astype(jnp.float32).reshape(NW, 12)
    th = jnp.pad(th, ((0, 0), (0, 4)))
    out = _resample(packed, th)
    return out.reshape(B, P, H, W, D, C)
